# baseline (speedup 1.0000x reference)
import functools

import jax
import jax.numpy as jnp
from jax import lax
from jax.experimental import pallas as pl
from jax.experimental.pallas import tpu as pltpu
from jax.experimental.pallas import tpu_sc as plsc

_NUM_CLASS = 150
_NBINS = _NUM_CLASS * _NUM_CLASS
_HIST_PAD = 22528
_NH = 4
_NC = 2
_NS = 16
_NW = _NC * _NS
_B, _H, _W = 16, 512, 512
_ROWS_W = _B * _H // _NW
_CROWS = 16
_NCHUNK = _ROWS_W // _CROWS
_CPIX = _CROWS * _W
_L = 16
_GROUPS = _CPIX // _L
_GPR = _W // _L


def _make_kernel():
    mesh = plsc.VectorSubcoreMesh(
        core_axis_name="c", subcore_axis_name="s",
        num_cores=_NC, num_subcores=_NS,
    )

    @functools.partial(
        pl.kernel,
        out_type=jax.ShapeDtypeStruct((_NW, _HIST_PAD), jnp.float32),
        mesh=mesh,
        scratch_types=[
            pltpu.VMEM((_CROWS, _W), jnp.int32),
            pltpu.VMEM((_CROWS, _W), jnp.int32),
            pltpu.VMEM((_CROWS, _W), jnp.int32),
            pltpu.VMEM((_CROWS, _W), jnp.int32),
            [pltpu.VMEM((_HIST_PAD,), jnp.float32)] * _NH,
            pltpu.SemaphoreType.DMA,
            pltpu.SemaphoreType.DMA,
        ],
        compiler_params=pltpu.CompilerParams(needs_layout_passes=False),
    )
    def cm_kernel(pred_hbm, label_hbm, out_hbm,
                  pred_v0, pred_v1, label_v0, label_v1, hists,
                  sem0, sem1):
        wid = lax.axis_index("s") * _NC + lax.axis_index("c")
        n = wid // 2
        row0 = (wid % 2) * _ROWS_W
        pred_bufs = (pred_v0, pred_v1)
        label_bufs = (label_v0, label_v1)
        sems = (sem0, sem1)

        zeros = jnp.zeros((_L,), jnp.float32)
        ones = jnp.ones((_L,), jnp.float32)

        @plsc.parallel_loop(0, _HIST_PAD // _L, unroll=4)
        def _zero(i):
            for j in range(_NH):
                hists[j][pl.ds(i * _L, _L)] = zeros

        for b in range(2):
            r = row0 + b * _CROWS
            pltpu.async_copy(pred_hbm.at[n, pl.ds(r, _CROWS), :],
                             pred_bufs[b], sems[b])
            pltpu.async_copy(label_hbm.at[n, pl.ds(r, _CROWS), :],
                             label_bufs[b], sems[b])

        def outer(g, carry):
            for b in range(2):
                ci = 2 * g + b
                r = row0 + ci * _CROWS
                pltpu.make_async_copy(pred_hbm.at[n, pl.ds(r, _CROWS), :],
                                      pred_bufs[b], sems[b]).wait()
                pltpu.make_async_copy(label_hbm.at[n, pl.ds(r, _CROWS), :],
                                      label_bufs[b], sems[b]).wait()

                pv, lv = pred_bufs[b], label_bufs[b]

                @plsc.parallel_loop(0, _GROUPS, step=_NH, unroll=4)
                def _pix(i):
                    for j in range(_NH):
                        gi = i + j
                        row = gi // _GPR
                        col = (gi % _GPR) * _L
                        p = pv[row, pl.ds(col, _L)]
                        lbl = lv[row, pl.ds(col, _L)]
                        mask = (lbl >= 0) & (lbl < _NUM_CLASS)
                        bins = lbl * _NUM_CLASS + p
                        plsc.addupdate_scatter(hists[j], [bins], ones,
                                               mask=mask)

                @pl.when(ci + 2 < _NCHUNK)
                def _prefetch():
                    r2 = row0 + (ci + 2) * _CROWS
                    pltpu.async_copy(pred_hbm.at[n, pl.ds(r2, _CROWS), :],
                                     pred_bufs[b], sems[b])
                    pltpu.async_copy(label_hbm.at[n, pl.ds(r2, _CROWS), :],
                                     label_bufs[b], sems[b])
            return carry

        lax.fori_loop(0, _NCHUNK // 2, outer, 0)

        @plsc.parallel_loop(0, _HIST_PAD // _L, unroll=4)
        def _merge(i):
            at = i * _L
            s = hists[0][pl.ds(at, _L)]
            for j in range(1, _NH):
                s = s + hists[j][pl.ds(at, _L)]
            hists[0][pl.ds(at, _L)] = s

        pltpu.sync_copy(hists[0], out_hbm.at[wid])

    return cm_kernel


_cm_kernel = _make_kernel()


@jax.jit
def kernel(imgPredict, imgLabel, confusionMatrix):
    parts = _cm_kernel(imgPredict, imgLabel)
    cm = parts.sum(axis=0)[:_NBINS].reshape(_NUM_CLASS, _NUM_CLASS)
    return confusionMatrix + cm


# speedup vs baseline: 1.0785x; 1.0785x over previous
import functools

import jax
import jax.numpy as jnp
from jax import lax
from jax.experimental import pallas as pl
from jax.experimental.pallas import tpu as pltpu
from jax.experimental.pallas import tpu_sc as plsc

_NUM_CLASS = 150
_NBINS = _NUM_CLASS * _NUM_CLASS
_HIST_PAD = 22528
_NH = 2
_NC = 2
_NS = 16
_NW = _NC * _NS
_B, _H, _W = 16, 512, 512
_ROWS_W = _B * _H // _NW
_CROWS = 16
_NCHUNK = _ROWS_W // _CROWS
_CPIX = _CROWS * _W
_L = 16
_GROUPS = _CPIX // _L
_GPR = _W // _L


def _make_kernel():
    mesh = plsc.VectorSubcoreMesh(
        core_axis_name="c", subcore_axis_name="s",
        num_cores=_NC, num_subcores=_NS,
    )

    @functools.partial(
        pl.kernel,
        out_type=jax.ShapeDtypeStruct((_NW, _HIST_PAD), jnp.float32),
        mesh=mesh,
        scratch_types=[
            pltpu.VMEM((_CROWS, _W), jnp.int32),
            pltpu.VMEM((_CROWS, _W), jnp.int32),
            pltpu.VMEM((_CROWS, _W), jnp.int32),
            pltpu.VMEM((_CROWS, _W), jnp.int32),
            [pltpu.VMEM((_HIST_PAD,), jnp.float32)] * _NH,
            pltpu.SemaphoreType.DMA,
            pltpu.SemaphoreType.DMA,
        ],
        compiler_params=pltpu.CompilerParams(needs_layout_passes=False),
    )
    def cm_kernel(pred_hbm, label_hbm, out_hbm,
                  pred_v0, pred_v1, label_v0, label_v1, hists,
                  sem0, sem1):
        wid = lax.axis_index("s") * _NC + lax.axis_index("c")
        n = wid // 2
        row0 = (wid % 2) * _ROWS_W
        pred_bufs = (pred_v0, pred_v1)
        label_bufs = (label_v0, label_v1)
        sems = (sem0, sem1)

        zeros = jnp.zeros((_L,), jnp.float32)
        ones = jnp.ones((_L,), jnp.float32)

        @plsc.parallel_loop(0, _HIST_PAD // _L, unroll=4)
        def _zero(i):
            for j in range(_NH):
                hists[j][pl.ds(i * _L, _L)] = zeros

        for b in range(2):
            r = row0 + b * _CROWS
            pltpu.async_copy(pred_hbm.at[n, pl.ds(r, _CROWS), :],
                             pred_bufs[b], sems[b])
            pltpu.async_copy(label_hbm.at[n, pl.ds(r, _CROWS), :],
                             label_bufs[b], sems[b])

        def outer(g, carry):
            for b in range(2):
                ci = 2 * g + b
                r = row0 + ci * _CROWS
                pltpu.make_async_copy(pred_hbm.at[n, pl.ds(r, _CROWS), :],
                                      pred_bufs[b], sems[b]).wait()
                pltpu.make_async_copy(label_hbm.at[n, pl.ds(r, _CROWS), :],
                                      label_bufs[b], sems[b]).wait()

                pv, lv = pred_bufs[b], label_bufs[b]

                @plsc.parallel_loop(0, _GROUPS, step=_NH, unroll=4)
                def _pix(i):
                    for j in range(_NH):
                        gi = i + j
                        row = gi // _GPR
                        col = (gi % _GPR) * _L
                        p = pv[row, pl.ds(col, _L)]
                        lbl = lv[row, pl.ds(col, _L)]
                        mask = (lbl >= 0) & (lbl < _NUM_CLASS)
                        bins = lbl * _NUM_CLASS + p
                        plsc.addupdate_scatter(hists[j], [bins], ones,
                                               mask=mask)

                @pl.when(ci + 2 < _NCHUNK)
                def _prefetch():
                    r2 = row0 + (ci + 2) * _CROWS
                    pltpu.async_copy(pred_hbm.at[n, pl.ds(r2, _CROWS), :],
                                     pred_bufs[b], sems[b])
                    pltpu.async_copy(label_hbm.at[n, pl.ds(r2, _CROWS), :],
                                     label_bufs[b], sems[b])
            return carry

        lax.fori_loop(0, _NCHUNK // 2, outer, 0)

        @plsc.parallel_loop(0, _HIST_PAD // _L, unroll=4)
        def _merge(i):
            at = i * _L
            s = hists[0][pl.ds(at, _L)]
            for j in range(1, _NH):
                s = s + hists[j][pl.ds(at, _L)]
            hists[0][pl.ds(at, _L)] = s

        pltpu.sync_copy(hists[0], out_hbm.at[wid])

    return cm_kernel


_cm_kernel = _make_kernel()


@jax.jit
def kernel(imgPredict, imgLabel, confusionMatrix):
    parts = _cm_kernel(imgPredict, imgLabel)
    cm = parts.sum(axis=0)[:_NBINS].reshape(_NUM_CLASS, _NUM_CLASS)
    return confusionMatrix + cm


# speedup vs baseline: 1.1118x; 1.0309x over previous
import functools

import jax
import jax.numpy as jnp
from jax import lax
from jax.experimental import pallas as pl
from jax.experimental.pallas import tpu as pltpu
from jax.experimental.pallas import tpu_sc as plsc

_NUM_CLASS = 150
_NBINS = _NUM_CLASS * _NUM_CLASS
_HIST_PAD = 22528
_NH = 2
_NC = 2
_NS = 16
_NW = _NC * _NS
_B, _H, _W = 16, 512, 512
_ROWS_W = _B * _H // _NW
_CROWS = 16
_NCHUNK = _ROWS_W // _CROWS
_CPIX = _CROWS * _W
_L = 16
_GROUPS = _CPIX // _L
_GPR = _W // _L


def _make_kernel():
    mesh = plsc.VectorSubcoreMesh(
        core_axis_name="c", subcore_axis_name="s",
        num_cores=_NC, num_subcores=_NS,
    )

    @functools.partial(
        pl.kernel,
        out_type=jax.ShapeDtypeStruct((_NW, _HIST_PAD), jnp.float32),
        mesh=mesh,
        scratch_types=[
            pltpu.VMEM((_CROWS, _W), jnp.int32),
            pltpu.VMEM((_CROWS, _W), jnp.int32),
            pltpu.VMEM((_CROWS, _W), jnp.int32),
            pltpu.VMEM((_CROWS, _W), jnp.int32),
            [pltpu.VMEM((_HIST_PAD,), jnp.float32)] * _NH,
            pltpu.SemaphoreType.DMA,
            pltpu.SemaphoreType.DMA,
        ],
        compiler_params=pltpu.CompilerParams(needs_layout_passes=False),
    )
    def cm_kernel(pred_hbm, label_hbm, out_hbm,
                  pred_v0, pred_v1, label_v0, label_v1, hists,
                  sem0, sem1):
        wid = lax.axis_index("s") * _NC + lax.axis_index("c")
        n = wid // 2
        row0 = (wid % 2) * _ROWS_W
        pred_bufs = (pred_v0, pred_v1)
        label_bufs = (label_v0, label_v1)
        sems = (sem0, sem1)

        zeros = jnp.zeros((_L,), jnp.float32)
        ones = jnp.ones((_L,), jnp.float32)

        for b in range(2):
            r = row0 + b * _CROWS
            pltpu.async_copy(pred_hbm.at[n, pl.ds(r, _CROWS), :],
                             pred_bufs[b], sems[b])
            pltpu.async_copy(label_hbm.at[n, pl.ds(r, _CROWS), :],
                             label_bufs[b], sems[b])

        @plsc.parallel_loop(0, _HIST_PAD // _L, unroll=4)
        def _zero(i):
            for j in range(_NH):
                hists[j][pl.ds(i * _L, _L)] = zeros

        def outer(g, carry):
            for b in range(2):
                ci = 2 * g + b
                r = row0 + ci * _CROWS
                pltpu.make_async_copy(pred_hbm.at[n, pl.ds(r, _CROWS), :],
                                      pred_bufs[b], sems[b]).wait()
                pltpu.make_async_copy(label_hbm.at[n, pl.ds(r, _CROWS), :],
                                      label_bufs[b], sems[b]).wait()

                pv, lv = pred_bufs[b], label_bufs[b]

                @plsc.parallel_loop(0, _GROUPS, step=_NH, unroll=4)
                def _pix(i):
                    for j in range(_NH):
                        gi = i + j
                        row = gi // _GPR
                        col = (gi % _GPR) * _L
                        p = pv[row, pl.ds(col, _L)]
                        lbl = lv[row, pl.ds(col, _L)]
                        mask = (lbl >= 0) & (lbl < _NUM_CLASS)
                        bins = lbl * _NUM_CLASS + p
                        plsc.addupdate_scatter(hists[j], [bins], ones,
                                               mask=mask)

                @pl.when(ci + 2 < _NCHUNK)
                def _prefetch():
                    r2 = row0 + (ci + 2) * _CROWS
                    pltpu.async_copy(pred_hbm.at[n, pl.ds(r2, _CROWS), :],
                                     pred_bufs[b], sems[b])
                    pltpu.async_copy(label_hbm.at[n, pl.ds(r2, _CROWS), :],
                                     label_bufs[b], sems[b])
            return carry

        lax.fori_loop(0, _NCHUNK // 2, outer, 0)

        @plsc.parallel_loop(0, _HIST_PAD // _L, unroll=4)
        def _merge(i):
            at = i * _L
            s = hists[0][pl.ds(at, _L)]
            for j in range(1, _NH):
                s = s + hists[j][pl.ds(at, _L)]
            hists[0][pl.ds(at, _L)] = s

        pltpu.sync_copy(hists[0], out_hbm.at[wid])

    return cm_kernel


_cm_kernel = _make_kernel()


@jax.jit
def kernel(imgPredict, imgLabel, confusionMatrix):
    parts = _cm_kernel(imgPredict, imgLabel)
    cm = parts.sum(axis=0)[:_NBINS].reshape(_NUM_CLASS, _NUM_CLASS)
    return confusionMatrix + cm


# speedup vs baseline: 1.1584x; 1.0420x over previous
import functools

import jax
import jax.numpy as jnp
from jax import lax
from jax.experimental import pallas as pl
from jax.experimental.pallas import tpu as pltpu
from jax.experimental.pallas import tpu_sc as plsc

_NUM_CLASS = 150
_NBINS = _NUM_CLASS * _NUM_CLASS
_HIST_PAD = 22528
_NH = 1
_NC = 2
_NS = 16
_NW = _NC * _NS
_B, _H, _W = 16, 512, 512
_ROWS_W = _B * _H // _NW
_CROWS = 16
_NCHUNK = _ROWS_W // _CROWS
_CPIX = _CROWS * _W
_L = 16
_GROUPS = _CPIX // _L
_GPR = _W // _L


def _make_kernel():
    mesh = plsc.VectorSubcoreMesh(
        core_axis_name="c", subcore_axis_name="s",
        num_cores=_NC, num_subcores=_NS,
    )

    @functools.partial(
        pl.kernel,
        out_type=jax.ShapeDtypeStruct((_NW, _HIST_PAD), jnp.float32),
        mesh=mesh,
        scratch_types=[
            pltpu.VMEM((_CROWS, _W), jnp.int32),
            pltpu.VMEM((_CROWS, _W), jnp.int32),
            pltpu.VMEM((_CROWS, _W), jnp.int32),
            pltpu.VMEM((_CROWS, _W), jnp.int32),
            [pltpu.VMEM((_HIST_PAD,), jnp.float32)] * _NH,
            pltpu.SemaphoreType.DMA,
            pltpu.SemaphoreType.DMA,
        ],
        compiler_params=pltpu.CompilerParams(needs_layout_passes=False),
    )
    def cm_kernel(pred_hbm, label_hbm, out_hbm,
                  pred_v0, pred_v1, label_v0, label_v1, hists,
                  sem0, sem1):
        wid = lax.axis_index("s") * _NC + lax.axis_index("c")
        n = wid // 2
        row0 = (wid % 2) * _ROWS_W
        pred_bufs = (pred_v0, pred_v1)
        label_bufs = (label_v0, label_v1)
        sems = (sem0, sem1)

        zeros = jnp.zeros((_L,), jnp.float32)
        ones = jnp.ones((_L,), jnp.float32)

        for b in range(2):
            r = row0 + b * _CROWS
            pltpu.async_copy(pred_hbm.at[n, pl.ds(r, _CROWS), :],
                             pred_bufs[b], sems[b])
            pltpu.async_copy(label_hbm.at[n, pl.ds(r, _CROWS), :],
                             label_bufs[b], sems[b])

        @plsc.parallel_loop(0, _HIST_PAD // _L, unroll=4)
        def _zero(i):
            for j in range(_NH):
                hists[j][pl.ds(i * _L, _L)] = zeros

        def outer(g, carry):
            for b in range(2):
                ci = 2 * g + b
                r = row0 + ci * _CROWS
                pltpu.make_async_copy(pred_hbm.at[n, pl.ds(r, _CROWS), :],
                                      pred_bufs[b], sems[b]).wait()
                pltpu.make_async_copy(label_hbm.at[n, pl.ds(r, _CROWS), :],
                                      label_bufs[b], sems[b]).wait()

                pv, lv = pred_bufs[b], label_bufs[b]

                @plsc.parallel_loop(0, _GROUPS, step=_NH, unroll=8)
                def _pix(i):
                    for j in range(_NH):
                        gi = i + j
                        row = gi // _GPR
                        col = (gi % _GPR) * _L
                        p = pv[row, pl.ds(col, _L)]
                        lbl = lv[row, pl.ds(col, _L)]
                        mask = (lbl >= 0) & (lbl < _NUM_CLASS)
                        bins = lbl * _NUM_CLASS + p
                        plsc.addupdate_scatter(hists[j], [bins], ones,
                                               mask=mask)

                @pl.when(ci + 2 < _NCHUNK)
                def _prefetch():
                    r2 = row0 + (ci + 2) * _CROWS
                    pltpu.async_copy(pred_hbm.at[n, pl.ds(r2, _CROWS), :],
                                     pred_bufs[b], sems[b])
                    pltpu.async_copy(label_hbm.at[n, pl.ds(r2, _CROWS), :],
                                     label_bufs[b], sems[b])
            return carry

        lax.fori_loop(0, _NCHUNK // 2, outer, 0)

        @plsc.parallel_loop(0, _HIST_PAD // _L, unroll=4)
        def _merge(i):
            at = i * _L
            s = hists[0][pl.ds(at, _L)]
            for j in range(1, _NH):
                s = s + hists[j][pl.ds(at, _L)]
            hists[0][pl.ds(at, _L)] = s

        pltpu.sync_copy(hists[0], out_hbm.at[wid])

    return cm_kernel


_cm_kernel = _make_kernel()


@jax.jit
def kernel(imgPredict, imgLabel, confusionMatrix):
    parts = _cm_kernel(imgPredict, imgLabel)
    cm = parts.sum(axis=0)[:_NBINS].reshape(_NUM_CLASS, _NUM_CLASS)
    return confusionMatrix + cm
